# baseline (speedup 1.0000x reference)
import functools

import jax
import jax.numpy as jnp
from jax import lax
from jax.experimental import pallas as pl
from jax.experimental.pallas import tpu as pltpu
from jax.experimental.pallas import tpu_sc as plsc

N = 10000
E = 320000
B = 200
NB = 50
DM = 128
NS_SEM = 16

NC = 2
NSC = 16
NW = NC * NSC
EPW = E // NW
CK = 80
NCHUNK = EPW // CK
STR = 624
TAIL = N - NSC * STR
NP = 10240
DSTR = NP // NSC

BLK = 1000
GRID = N // BLK
GB = BLK // NB


H = 2
NBUF = 2 * H
NG = NCHUNK // H
NPAIR = NG // 2
NTAIL = NCHUNK - NG * H


def _sc_body(with_deg, *refs):
    it = iter(refs)
    z_hbm, src_hbm, dst_hbm, znd_hbm = next(it), next(it), next(it), next(it)
    if with_deg:
        zn_hbm, ones_hbm = next(it), next(it)
    part_hbm = next(it)
    if with_deg:
        degp_hbm = next(it)
    sidx = [next(it) for _ in range(NBUF)]
    didx = [next(it) for _ in range(NBUF)]
    rows = [next(it) for _ in range(NBUF)]
    if with_deg:
        onesv = next(it)
    acc = next(it)
    if with_deg:
        dacc = next(it)
    gsem = next(it)
    ssem = next(it)
    if with_deg:
        dsem = next(it)

    c = lax.axis_index("c")
    s = lax.axis_index("s")
    w = s * NC + c
    pltpu.sync_copy(znd_hbm.at[pl.ds(s * STR, STR)], acc.at[pl.ds(s * STR, STR)])

    @pl.when(s == NSC - 1)
    def _():
        pltpu.sync_copy(znd_hbm.at[pl.ds(NSC * STR, TAIL)],
                        acc.at[pl.ds(NSC * STR, TAIL)])
    if with_deg:
        pltpu.sync_copy(zn_hbm.at[pl.ds(s * DSTR, DSTR)], dacc.at[pl.ds(s * DSTR, DSTR)])
        pltpu.sync_copy(ones_hbm, onesv)
    plsc.subcore_barrier()

    def idx_load(g, base_buf):
        for k in range(H):
            base = w * EPW + (g * H + k) * CK
            pltpu.sync_copy(src_hbm.at[pl.ds(base, CK)], sidx[base_buf + k])
            pltpu.sync_copy(dst_hbm.at[pl.ds(base, CK)], didx[base_buf + k])

    def gathers_start(base_buf):
        for k in range(H):
            pltpu.async_copy(z_hbm.at[sidx[base_buf + k]],
                             rows[base_buf + k], gsem)

    def gathers_drain(base_buf):
        for k in range(H):
            pltpu.make_async_copy(z_hbm.at[sidx[base_buf + k]],
                                  rows[base_buf + k], gsem).wait()

    def scatters_start(base_buf):
        for k in range(H):
            pltpu.async_copy(rows[base_buf + k],
                             acc.at[didx[base_buf + k]], ssem, add=True)
            if with_deg:
                pltpu.async_copy(onesv, dacc.at[didx[base_buf + k]],
                                 dsem, add=True)

    def scatters_drain(base_buf):
        for k in range(H):
            pltpu.make_async_copy(rows[base_buf + k],
                                  acc.at[didx[base_buf + k]], ssem).wait()
            if with_deg:
                pltpu.make_async_copy(onesv, dacc.at[didx[base_buf + k]],
                                      dsem).wait()

    idx_load(0, 0)
    gathers_start(0)

    def pair(jp, carry):
        gA = 2 * jp
        idx_load(gA + 1, H)
        gathers_drain(0)
        scatters_start(0)
        gathers_start(H)
        scatters_drain(0)

        @pl.when(jp < NPAIR - 1)
        def _():
            idx_load(gA + 2, 0)
        gathers_drain(H)
        scatters_start(H)

        @pl.when(jp < NPAIR - 1)
        def _():
            gathers_start(0)
        scatters_drain(H)
        return carry

    lax.fori_loop(0, NPAIR, pair, 0)
    for t in range(NTAIL):
        base = w * EPW + (NG * H + t) * CK
        pltpu.sync_copy(src_hbm.at[pl.ds(base, CK)], sidx[0])
        pltpu.sync_copy(dst_hbm.at[pl.ds(base, CK)], didx[0])
        pltpu.async_copy(z_hbm.at[sidx[0]], rows[0], gsem).wait()
        pltpu.sync_copy(rows[0], acc.at[didx[0]], add=True)
        if with_deg:
            pltpu.sync_copy(onesv, dacc.at[didx[0]], add=True)
    plsc.subcore_barrier()
    pltpu.sync_copy(acc.at[pl.ds(s * STR, STR)], part_hbm.at[c, pl.ds(s * STR, STR)])

    @pl.when(s == NSC - 1)
    def _():
        pltpu.sync_copy(acc.at[pl.ds(NSC * STR, TAIL)],
                        part_hbm.at[c, pl.ds(NSC * STR, TAIL)])
    if with_deg:
        pltpu.sync_copy(dacc.at[pl.ds(s * DSTR, DSTR)], degp_hbm.at[c, pl.ds(s * DSTR, DSTR)])


def _sc_mesh():
    return plsc.VectorSubcoreMesh(core_axis_name="c", subcore_axis_name="s")


def _sc_pass_deg(z, src, dst, znd, zn, ones_k):
    scratch = (
        [pltpu.VMEM((CK,), jnp.int32)] * (2 * NBUF)
        + [pltpu.VMEM((CK, DM), jnp.float32)] * NBUF
        + [pltpu.VMEM((CK,), jnp.float32),
           pltpu.VMEM_SHARED((N, DM), jnp.float32),
           pltpu.VMEM_SHARED((NP,), jnp.float32)]
        + [pltpu.SemaphoreType.DMA] * 3
    )
    f = pl.kernel(
        functools.partial(_sc_body, True),
        out_type=(jax.ShapeDtypeStruct((NC, N, DM), jnp.float32),
                  jax.ShapeDtypeStruct((NC, NP), jnp.float32)),
        mesh=_sc_mesh(),
        scratch_types=scratch,
    )
    return f(z, src, dst, znd, zn, ones_k)


def _sc_pass(z, src, dst, znd):
    scratch = (
        [pltpu.VMEM((CK,), jnp.int32)] * (2 * NBUF)
        + [pltpu.VMEM((CK, DM), jnp.float32)] * NBUF
        + [pltpu.VMEM_SHARED((N, DM), jnp.float32)]
        + [pltpu.SemaphoreType.DMA] * 2
    )
    f = pl.kernel(
        functools.partial(_sc_body, False),
        out_type=jax.ShapeDtypeStruct((NC, N, DM), jnp.float32),
        mesh=_sc_mesh(),
        scratch_types=scratch,
    )
    return f(z, src, dst, znd)


def _pool(h):
    return jnp.max(h.reshape(GB, NB, DM), axis=1)[None]


def _k0_body(geo_ref, sem_ref, a_ref, ts_ref, p_ref, c_ref, wb_ref,
             h_ref, z_ref, g_ref):
    onehot = (sem_ref[...] == lax.broadcasted_iota(
        jnp.int32, (BLK, NS_SEM), 1)).astype(jnp.float32)
    h = (geo_ref[...] @ a_ref[...] + onehot @ ts_ref[...]
         + p_ref[...] + c_ref[...])
    h = jnp.maximum(h, 0.0)
    h_ref[...] = h
    z_ref[...] = h @ wb_ref[...]
    g_ref[...] = _pool(h)


def _gspec():
    return pl.BlockSpec((1, GB, DM), lambda i: (i, 0, 0))


def _gshape():
    return jax.ShapeDtypeStruct((GRID, GB, DM), jnp.float32)


def _tc_input_layer(geo_p, sem2, a_p, ts_p, p_tile, cvec, wb1):
    full = lambda shp: pl.BlockSpec(shp, lambda i: (0, 0))
    return pl.pallas_call(
        _k0_body,
        grid=(GRID,),
        in_specs=[
            pl.BlockSpec((BLK, 8), lambda i: (i, 0)),
            pl.BlockSpec((BLK, 1), lambda i: (i, 0)),
            full((8, DM)),
            full((NS_SEM, DM)),
            full((BLK, DM)),
            full((1, DM)),
            full((DM, DM)),
        ],
        out_specs=[
            pl.BlockSpec((BLK, DM), lambda i: (i, 0)),
            pl.BlockSpec((BLK, DM), lambda i: (i, 0)),
            _gspec(),
        ],
        out_shape=[
            jax.ShapeDtypeStruct((N, DM), jnp.float32),
            jax.ShapeDtypeStruct((N, DM), jnp.float32),
            _gshape(),
        ],
    )(geo_p, sem2, a_p, ts_p, p_tile, cvec, wb1)


def _combine(h_ref, p0_ref, p1_ref, d0_ref, d1_ref, wt_ref, b_ref):
    dsum = d0_ref[...] + d1_ref[...]
    ssum = p0_ref[...] + p1_ref[...]
    inv = 1.0 / jnp.maximum(dsum, 1.0)
    pre = h_ref[...] @ wt_ref[...] + ssum * inv + b_ref[...]
    return jnp.maximum(jnp.where(dsum > 0.0, pre, 0.0), 0.0)


def _klayer_body(h_ref, p0_ref, p1_ref, d0_ref, d1_ref,
                 wt_ref, b_ref, wb_ref, hout_ref, zout_ref, g_ref):
    hnew = _combine(h_ref, p0_ref, p1_ref, d0_ref, d1_ref, wt_ref, b_ref)
    hout_ref[...] = hnew
    zout_ref[...] = hnew @ wb_ref[...]
    g_ref[...] = _pool(hnew)


def _tc_layer(h, p0, p1, d0, d1, wt, b, wb):
    full2 = lambda shp: pl.BlockSpec(shp, lambda i: (0, 0))
    nblk = lambda: pl.BlockSpec((BLK, DM), lambda i: (i, 0))
    return pl.pallas_call(
        _klayer_body,
        grid=(GRID,),
        in_specs=[
            nblk(), nblk(), nblk(),
            pl.BlockSpec((BLK, 1), lambda i: (i, 0)),
            pl.BlockSpec((BLK, 1), lambda i: (i, 0)),
            full2((DM, DM)),
            full2((1, DM)),
            full2((DM, DM)),
        ],
        out_specs=[nblk(), nblk(), _gspec()],
        out_shape=[
            jax.ShapeDtypeStruct((N, DM), jnp.float32),
            jax.ShapeDtypeStruct((N, DM), jnp.float32),
            _gshape(),
        ],
    )(h, p0, p1, d0, d1, wt, b, wb)


def _k3_body(h_ref, p0_ref, p1_ref, d0_ref, d1_ref, wt_ref, b_ref,
             g0_ref, g1_ref, g2_ref, w0_ref, w1_ref, w2_ref, w3_ref,
             bagg_ref, wmu_ref, bmu_ref, wvar_ref, bvar_ref,
             mu_ref, lv_ref):
    h3 = _combine(h_ref, p0_ref, p1_ref, d0_ref, d1_ref, wt_ref, b_ref)
    g3 = _pool(h3)[0]
    latent = (g0_ref[0] @ w0_ref[...] + g1_ref[0] @ w1_ref[...]
              + g2_ref[0] @ w2_ref[...] + g3 @ w3_ref[...] + bagg_ref[...])
    mu_ref[...] = (latent @ wmu_ref[...] + bmu_ref[...])[None]
    lv_ref[...] = (latent @ wvar_ref[...] + bvar_ref[...])[None]


def _tc_final_layer(h, p0, p1, d0, d1, wt, b, g0, g1, g2,
                    w0, w1, w2, w3, bagg2, Wmu, bmu2, Wvar, bvar2):
    full2 = lambda shp: pl.BlockSpec(shp, lambda i: (0, 0))
    nblk = lambda: pl.BlockSpec((BLK, DM), lambda i: (i, 0))
    mu, lv = pl.pallas_call(
        _k3_body,
        grid=(GRID,),
        in_specs=[
            nblk(), nblk(), nblk(),
            pl.BlockSpec((BLK, 1), lambda i: (i, 0)),
            pl.BlockSpec((BLK, 1), lambda i: (i, 0)),
            full2((DM, DM)),
            full2((1, DM)),
            _gspec(), _gspec(), _gspec(),
            full2((DM, DM)), full2((DM, DM)), full2((DM, DM)), full2((DM, DM)),
            full2((1, DM)),
            full2((DM, DM)), full2((1, DM)),
            full2((DM, DM)), full2((1, DM)),
        ],
        out_specs=[_gspec(), _gspec()],
        out_shape=[_gshape(), _gshape()],
    )(h, p0, p1, d0, d1, wt, b, g0, g1, g2,
      w0, w1, w2, w3, bagg2, Wmu, bmu2, Wvar, bvar2)
    return mu.reshape(B, DM), lv.reshape(B, DM)


def kernel(geometry, semantic, edge_index, batch, ptr, Wg, bg, emb, Wlot,
           blot, W1, b1, W2, b2, W3, b3, Wagg, bagg, Wmu, bmu, Wvar, bvar):
    f32 = jnp.float32
    a_p = jnp.pad(Wg @ Wlot[:DM], ((0, 3), (0, 0)))
    ts_p = jnp.pad(emb @ Wlot[DM:2 * DM], ((0, 5), (0, 0)))
    p_tile = jnp.tile(Wlot[2 * DM:], (GB, 1))
    cvec = (bg @ Wlot[:DM] + blot)[None, :]
    geo_p = jnp.pad(geometry, ((0, 0), (0, 3)))
    sem2 = semantic.reshape(N, 1)
    src = edge_index[0]
    dst = edge_index[1]
    znd = jnp.zeros((N, DM), f32)
    zn = jnp.zeros((NP,), f32)
    ones_k = jnp.ones((CK,), f32)

    h0, z1, g0 = _tc_input_layer(geo_p, sem2, a_p, ts_p, p_tile, cvec, W1[DM:])

    part1, degp = _sc_pass_deg(z1, src, dst, znd, zn, ones_k)
    d0 = degp[0, :N].reshape(N, 1)
    d1 = degp[1, :N].reshape(N, 1)

    h1, z2, g1 = _tc_layer(h0, part1[0], part1[1], d0, d1,
                           W1[:DM], b1[None, :], W2[DM:])
    part2 = _sc_pass(z2, src, dst, znd)
    h2, z3, g2 = _tc_layer(h1, part2[0], part2[1], d0, d1,
                           W2[:DM], b2[None, :], W3[DM:])
    part3 = _sc_pass(z3, src, dst, znd)
    mu, lv = _tc_final_layer(h2, part3[0], part3[1], d0, d1,
                             W3[:DM], b3[None, :], g0, g1, g2,
                             Wagg[:DM], Wagg[DM:2 * DM], Wagg[2 * DM:3 * DM],
                             Wagg[3 * DM:], bagg[None, :], Wmu, bmu[None, :],
                             Wvar, bvar[None, :])
    return (mu, lv)


# speedup vs baseline: 1.2460x; 1.2460x over previous
import functools

import jax
import jax.numpy as jnp
from jax import lax
from jax.experimental import pallas as pl
from jax.experimental.pallas import tpu as pltpu
from jax.experimental.pallas import tpu_sc as plsc

N = 10000
E = 320000
B = 200
NB = 50
DM = 128
NS_SEM = 16

NC = 2
NSC = 16
NW = NC * NSC
EPW = E // NW
CK = 80
NCHUNK = EPW // CK
STR = 624
TAIL = N - NSC * STR
NP = 10240
DSTR = NP // NSC

BLK = 1000
GRID = N // BLK
GB = BLK // NB


NBUF = 4
NOUT = NCHUNK // NBUF
NTAIL = NCHUNK % NBUF


def _sc_body(with_deg, *refs):
    it = iter(refs)
    z_hbm, src_hbm, dst_hbm, znd_hbm = next(it), next(it), next(it), next(it)
    if with_deg:
        zn_hbm, ones_hbm = next(it), next(it)
    part_hbm = next(it)
    if with_deg:
        degp_hbm = next(it)
    sidx = [next(it) for _ in range(NBUF)]
    didx = [next(it) for _ in range(NBUF)]
    rows = [next(it) for _ in range(NBUF)]
    if with_deg:
        onesv = next(it)
    acc = next(it)
    if with_deg:
        dacc = next(it)
    isem = [next(it) for _ in range(NBUF)]
    gsem = [next(it) for _ in range(NBUF)]
    ssem = [next(it) for _ in range(NBUF)]
    if with_deg:
        dsem = [next(it) for _ in range(NBUF)]

    c = lax.axis_index("c")
    s = lax.axis_index("s")
    w = s * NC + c
    pltpu.sync_copy(znd_hbm.at[pl.ds(s * STR, STR)], acc.at[pl.ds(s * STR, STR)])

    @pl.when(s == NSC - 1)
    def _():
        pltpu.sync_copy(znd_hbm.at[pl.ds(NSC * STR, TAIL)],
                        acc.at[pl.ds(NSC * STR, TAIL)])
    if with_deg:
        pltpu.sync_copy(zn_hbm.at[pl.ds(s * DSTR, DSTR)], dacc.at[pl.ds(s * DSTR, DSTR)])
        pltpu.sync_copy(ones_hbm, onesv)
    plsc.subcore_barrier()

    def drain_scatters(b):
        pltpu.make_async_copy(rows[b], acc.at[didx[b]], ssem[b]).wait()
        if with_deg:
            pltpu.make_async_copy(onesv, dacc.at[didx[b]], dsem[b]).wait()

    def outer(j0, carry):
        jbase = w * EPW + j0 * (NBUF * CK)
        idesc = []
        for b in range(NBUF):
            @pl.when(j0 > 0)
            def _(b=b):
                drain_scatters(b)
            idesc.append((
                pltpu.async_copy(src_hbm.at[pl.ds(jbase + b * CK, CK)],
                                 sidx[b], isem[b]),
                pltpu.async_copy(dst_hbm.at[pl.ds(jbase + b * CK, CK)],
                                 didx[b], isem[b]),
            ))
        gdesc = []
        for b in range(NBUF):
            idesc[b][0].wait()
            idesc[b][1].wait()
            gdesc.append(pltpu.async_copy(z_hbm.at[sidx[b]], rows[b], gsem[b]))
        for b in range(NBUF):
            gdesc[b].wait()
            pltpu.async_copy(rows[b], acc.at[didx[b]], ssem[b], add=True)
            if with_deg:
                pltpu.async_copy(onesv, dacc.at[didx[b]], dsem[b], add=True)
        return carry

    lax.fori_loop(0, NOUT, outer, 0)
    for b in range(NBUF):
        drain_scatters(b)
    for t in range(NTAIL):
        base = w * EPW + (NOUT * NBUF + t) * CK
        pltpu.sync_copy(src_hbm.at[pl.ds(base, CK)], sidx[t])
        pltpu.sync_copy(dst_hbm.at[pl.ds(base, CK)], didx[t])
        pltpu.async_copy(z_hbm.at[sidx[t]], rows[t], gsem[t]).wait()
        pltpu.async_copy(rows[t], acc.at[didx[t]], ssem[t], add=True)
        if with_deg:
            pltpu.async_copy(onesv, dacc.at[didx[t]], dsem[t], add=True)
    for t in range(NTAIL):
        drain_scatters(t)
    plsc.subcore_barrier()
    pltpu.sync_copy(acc.at[pl.ds(s * STR, STR)], part_hbm.at[c, pl.ds(s * STR, STR)])

    @pl.when(s == NSC - 1)
    def _():
        pltpu.sync_copy(acc.at[pl.ds(NSC * STR, TAIL)],
                        part_hbm.at[c, pl.ds(NSC * STR, TAIL)])
    if with_deg:
        pltpu.sync_copy(dacc.at[pl.ds(s * DSTR, DSTR)], degp_hbm.at[c, pl.ds(s * DSTR, DSTR)])


def _sc_mesh():
    return plsc.VectorSubcoreMesh(core_axis_name="c", subcore_axis_name="s")


def _sc_pass_deg(z, src, dst, znd, zn, ones_k):
    scratch = (
        [pltpu.VMEM((CK,), jnp.int32)] * (2 * NBUF)
        + [pltpu.VMEM((CK, DM), jnp.float32)] * NBUF
        + [pltpu.VMEM((CK,), jnp.float32),
           pltpu.VMEM_SHARED((N, DM), jnp.float32),
           pltpu.VMEM_SHARED((NP,), jnp.float32)]
        + [pltpu.SemaphoreType.DMA] * (4 * NBUF)
    )
    f = pl.kernel(
        functools.partial(_sc_body, True),
        out_type=(jax.ShapeDtypeStruct((NC, N, DM), jnp.float32),
                  jax.ShapeDtypeStruct((NC, NP), jnp.float32)),
        mesh=_sc_mesh(),
        scratch_types=scratch,
    )
    return f(z, src, dst, znd, zn, ones_k)


def _sc_pass(z, src, dst, znd):
    scratch = (
        [pltpu.VMEM((CK,), jnp.int32)] * (2 * NBUF)
        + [pltpu.VMEM((CK, DM), jnp.float32)] * NBUF
        + [pltpu.VMEM_SHARED((N, DM), jnp.float32)]
        + [pltpu.SemaphoreType.DMA] * (3 * NBUF)
    )
    f = pl.kernel(
        functools.partial(_sc_body, False),
        out_type=jax.ShapeDtypeStruct((NC, N, DM), jnp.float32),
        mesh=_sc_mesh(),
        scratch_types=scratch,
    )
    return f(z, src, dst, znd)


def _pool(h):
    return jnp.max(h.reshape(GB, NB, DM), axis=1)[None]


def _k0_body(geo_ref, sem_ref, a_ref, ts_ref, p_ref, c_ref, wb_ref,
             h_ref, z_ref, g_ref):
    onehot = (sem_ref[...] == lax.broadcasted_iota(
        jnp.int32, (BLK, NS_SEM), 1)).astype(jnp.float32)
    h = (geo_ref[...] @ a_ref[...] + onehot @ ts_ref[...]
         + p_ref[...] + c_ref[...])
    h = jnp.maximum(h, 0.0)
    h_ref[...] = h
    z_ref[...] = h @ wb_ref[...]
    g_ref[...] = _pool(h)


def _gspec():
    return pl.BlockSpec((1, GB, DM), lambda i: (i, 0, 0))


def _gshape():
    return jax.ShapeDtypeStruct((GRID, GB, DM), jnp.float32)


def _tc_input_layer(geo_p, sem2, a_p, ts_p, p_tile, cvec, wb1):
    full = lambda shp: pl.BlockSpec(shp, lambda i: (0, 0))
    return pl.pallas_call(
        _k0_body,
        grid=(GRID,),
        in_specs=[
            pl.BlockSpec((BLK, 8), lambda i: (i, 0)),
            pl.BlockSpec((BLK, 1), lambda i: (i, 0)),
            full((8, DM)),
            full((NS_SEM, DM)),
            full((BLK, DM)),
            full((1, DM)),
            full((DM, DM)),
        ],
        out_specs=[
            pl.BlockSpec((BLK, DM), lambda i: (i, 0)),
            pl.BlockSpec((BLK, DM), lambda i: (i, 0)),
            _gspec(),
        ],
        out_shape=[
            jax.ShapeDtypeStruct((N, DM), jnp.float32),
            jax.ShapeDtypeStruct((N, DM), jnp.float32),
            _gshape(),
        ],
    )(geo_p, sem2, a_p, ts_p, p_tile, cvec, wb1)


def _combine(h_ref, p0_ref, p1_ref, d0_ref, d1_ref, wt_ref, b_ref):
    dsum = d0_ref[...] + d1_ref[...]
    ssum = p0_ref[...] + p1_ref[...]
    inv = 1.0 / jnp.maximum(dsum, 1.0)
    pre = h_ref[...] @ wt_ref[...] + ssum * inv + b_ref[...]
    return jnp.maximum(jnp.where(dsum > 0.0, pre, 0.0), 0.0)


def _klayer_body(h_ref, p0_ref, p1_ref, d0_ref, d1_ref,
                 wt_ref, b_ref, wb_ref, hout_ref, zout_ref, g_ref):
    hnew = _combine(h_ref, p0_ref, p1_ref, d0_ref, d1_ref, wt_ref, b_ref)
    hout_ref[...] = hnew
    zout_ref[...] = hnew @ wb_ref[...]
    g_ref[...] = _pool(hnew)


def _tc_layer(h, p0, p1, d0, d1, wt, b, wb):
    full2 = lambda shp: pl.BlockSpec(shp, lambda i: (0, 0))
    nblk = lambda: pl.BlockSpec((BLK, DM), lambda i: (i, 0))
    return pl.pallas_call(
        _klayer_body,
        grid=(GRID,),
        in_specs=[
            nblk(), nblk(), nblk(),
            pl.BlockSpec((BLK, 1), lambda i: (i, 0)),
            pl.BlockSpec((BLK, 1), lambda i: (i, 0)),
            full2((DM, DM)),
            full2((1, DM)),
            full2((DM, DM)),
        ],
        out_specs=[nblk(), nblk(), _gspec()],
        out_shape=[
            jax.ShapeDtypeStruct((N, DM), jnp.float32),
            jax.ShapeDtypeStruct((N, DM), jnp.float32),
            _gshape(),
        ],
    )(h, p0, p1, d0, d1, wt, b, wb)


def _k3_body(h_ref, p0_ref, p1_ref, d0_ref, d1_ref, wt_ref, b_ref,
             g0_ref, g1_ref, g2_ref, w0_ref, w1_ref, w2_ref, w3_ref,
             bagg_ref, wmu_ref, bmu_ref, wvar_ref, bvar_ref,
             mu_ref, lv_ref):
    h3 = _combine(h_ref, p0_ref, p1_ref, d0_ref, d1_ref, wt_ref, b_ref)
    g3 = _pool(h3)[0]
    latent = (g0_ref[0] @ w0_ref[...] + g1_ref[0] @ w1_ref[...]
              + g2_ref[0] @ w2_ref[...] + g3 @ w3_ref[...] + bagg_ref[...])
    mu_ref[...] = (latent @ wmu_ref[...] + bmu_ref[...])[None]
    lv_ref[...] = (latent @ wvar_ref[...] + bvar_ref[...])[None]


def _tc_final_layer(h, p0, p1, d0, d1, wt, b, g0, g1, g2,
                    w0, w1, w2, w3, bagg2, Wmu, bmu2, Wvar, bvar2):
    full2 = lambda shp: pl.BlockSpec(shp, lambda i: (0, 0))
    nblk = lambda: pl.BlockSpec((BLK, DM), lambda i: (i, 0))
    mu, lv = pl.pallas_call(
        _k3_body,
        grid=(GRID,),
        in_specs=[
            nblk(), nblk(), nblk(),
            pl.BlockSpec((BLK, 1), lambda i: (i, 0)),
            pl.BlockSpec((BLK, 1), lambda i: (i, 0)),
            full2((DM, DM)),
            full2((1, DM)),
            _gspec(), _gspec(), _gspec(),
            full2((DM, DM)), full2((DM, DM)), full2((DM, DM)), full2((DM, DM)),
            full2((1, DM)),
            full2((DM, DM)), full2((1, DM)),
            full2((DM, DM)), full2((1, DM)),
        ],
        out_specs=[_gspec(), _gspec()],
        out_shape=[_gshape(), _gshape()],
    )(h, p0, p1, d0, d1, wt, b, g0, g1, g2,
      w0, w1, w2, w3, bagg2, Wmu, bmu2, Wvar, bvar2)
    return mu.reshape(B, DM), lv.reshape(B, DM)


def kernel(geometry, semantic, edge_index, batch, ptr, Wg, bg, emb, Wlot,
           blot, W1, b1, W2, b2, W3, b3, Wagg, bagg, Wmu, bmu, Wvar, bvar):
    f32 = jnp.float32
    a_p = jnp.pad(Wg @ Wlot[:DM], ((0, 3), (0, 0)))
    ts_p = jnp.pad(emb @ Wlot[DM:2 * DM], ((0, 5), (0, 0)))
    p_tile = jnp.tile(Wlot[2 * DM:], (GB, 1))
    cvec = (bg @ Wlot[:DM] + blot)[None, :]
    geo_p = jnp.pad(geometry, ((0, 0), (0, 3)))
    sem2 = semantic.reshape(N, 1)
    src = edge_index[0]
    dst = edge_index[1]
    znd = jnp.zeros((N, DM), f32)
    zn = jnp.zeros((NP,), f32)
    ones_k = jnp.ones((CK,), f32)

    h0, z1, g0 = _tc_input_layer(geo_p, sem2, a_p, ts_p, p_tile, cvec, W1[DM:])

    part1, degp = _sc_pass_deg(z1, src, dst, znd, zn, ones_k)
    d0 = degp[0, :N].reshape(N, 1)
    d1 = degp[1, :N].reshape(N, 1)

    h1, z2, g1 = _tc_layer(h0, part1[0], part1[1], d0, d1,
                           W1[:DM], b1[None, :], W2[DM:])
    part2 = _sc_pass(z2, src, dst, znd)
    h2, z3, g2 = _tc_layer(h1, part2[0], part2[1], d0, d1,
                           W2[:DM], b2[None, :], W3[DM:])
    part3 = _sc_pass(z3, src, dst, znd)
    mu, lv = _tc_final_layer(h2, part3[0], part3[1], d0, d1,
                             W3[:DM], b3[None, :], g0, g1, g2,
                             Wagg[:DM], Wagg[DM:2 * DM], Wagg[2 * DM:3 * DM],
                             Wagg[3 * DM:], bagg[None, :], Wmu, bmu[None, :],
                             Wvar, bvar[None, :])
    return (mu, lv)


# speedup vs baseline: 1.3728x; 1.1018x over previous
import functools

import jax
import jax.numpy as jnp
from jax import lax
from jax.experimental import pallas as pl
from jax.experimental.pallas import tpu as pltpu
from jax.experimental.pallas import tpu_sc as plsc

N = 10000
E = 320000
B = 200
NB = 50
DM = 128
NS_SEM = 16

NC = 2
NSC = 16
NW = NC * NSC
EPW = E // NW
CK = 80
NCHUNK = EPW // CK
STR = 624
TAIL = N - NSC * STR
NP = 10240
DSTR = NP // NSC

BLK = 1000
GRID = N // BLK
GB = BLK // NB


H = 2
NBUF = 2 * H
NG = NCHUNK // H
NPAIR = NG // 2
NTAIL = NCHUNK - NG * H


def _sc_body(with_deg, *refs):
    it = iter(refs)
    z_hbm, src_hbm, dst_hbm, znd_hbm = next(it), next(it), next(it), next(it)
    if with_deg:
        zn_hbm, ones_hbm = next(it), next(it)
    part_hbm = next(it)
    if with_deg:
        degp_hbm = next(it)
    sidx = [next(it) for _ in range(NBUF)]
    didx = [next(it) for _ in range(NBUF)]
    rows = [next(it) for _ in range(NBUF)]
    if with_deg:
        onesv = next(it)
    acc = next(it)
    if with_deg:
        dacc = next(it)
    isem = next(it)
    gsem = next(it)
    ssem = next(it)
    if with_deg:
        dsem = next(it)

    c = lax.axis_index("c")
    s = lax.axis_index("s")
    w = s * NC + c
    pltpu.sync_copy(znd_hbm.at[pl.ds(s * STR, STR)], acc.at[pl.ds(s * STR, STR)])

    @pl.when(s == NSC - 1)
    def _():
        pltpu.sync_copy(znd_hbm.at[pl.ds(NSC * STR, TAIL)],
                        acc.at[pl.ds(NSC * STR, TAIL)])
    if with_deg:
        pltpu.sync_copy(zn_hbm.at[pl.ds(s * DSTR, DSTR)], dacc.at[pl.ds(s * DSTR, DSTR)])
        pltpu.sync_copy(ones_hbm, onesv)
    plsc.subcore_barrier()

    def idx_start(g, base_buf):
        for k in range(H):
            base = w * EPW + (g * H + k) * CK
            pltpu.async_copy(src_hbm.at[pl.ds(base, CK)],
                             sidx[base_buf + k], isem)
            pltpu.async_copy(dst_hbm.at[pl.ds(base, CK)],
                             didx[base_buf + k], isem)

    def idx_wait(base_buf):
        for k in range(H):
            pltpu.make_async_copy(src_hbm.at[pl.ds(0, CK)],
                                  sidx[base_buf + k], isem).wait()
            pltpu.make_async_copy(dst_hbm.at[pl.ds(0, CK)],
                                  didx[base_buf + k], isem).wait()

    def gathers_start(base_buf):
        for k in range(H):
            pltpu.async_copy(z_hbm.at[sidx[base_buf + k]],
                             rows[base_buf + k], gsem)

    def gathers_drain(base_buf):
        for k in range(H):
            pltpu.make_async_copy(z_hbm.at[sidx[base_buf + k]],
                                  rows[base_buf + k], gsem).wait()

    def scatters_start(base_buf):
        for k in range(H):
            pltpu.async_copy(rows[base_buf + k],
                             acc.at[didx[base_buf + k]], ssem, add=True)
            if with_deg:
                pltpu.async_copy(onesv, dacc.at[didx[base_buf + k]],
                                 dsem, add=True)

    def scatters_drain(base_buf):
        for k in range(H):
            pltpu.make_async_copy(rows[base_buf + k],
                                  acc.at[didx[base_buf + k]], ssem).wait()
            if with_deg:
                pltpu.make_async_copy(onesv, dacc.at[didx[base_buf + k]],
                                      dsem).wait()

    idx_start(0, 0)
    idx_wait(0)
    gathers_start(0)

    def pair(jp, carry):
        gA = 2 * jp
        idx_start(gA + 1, H)
        gathers_drain(0)
        scatters_start(0)
        idx_wait(H)
        gathers_start(H)
        scatters_drain(0)

        @pl.when(jp < NPAIR - 1)
        def _():
            idx_start(gA + 2, 0)
        gathers_drain(H)
        scatters_start(H)

        @pl.when(jp < NPAIR - 1)
        def _():
            idx_wait(0)
            gathers_start(0)
        scatters_drain(H)
        return carry

    lax.fori_loop(0, NPAIR, pair, 0)
    for t in range(NTAIL):
        base = w * EPW + (NG * H + t) * CK
        pltpu.sync_copy(src_hbm.at[pl.ds(base, CK)], sidx[0])
        pltpu.sync_copy(dst_hbm.at[pl.ds(base, CK)], didx[0])
        pltpu.async_copy(z_hbm.at[sidx[0]], rows[0], gsem).wait()
        pltpu.sync_copy(rows[0], acc.at[didx[0]], add=True)
        if with_deg:
            pltpu.sync_copy(onesv, dacc.at[didx[0]], add=True)
    plsc.subcore_barrier()
    pltpu.sync_copy(acc.at[pl.ds(s * STR, STR)], part_hbm.at[c, pl.ds(s * STR, STR)])

    @pl.when(s == NSC - 1)
    def _():
        pltpu.sync_copy(acc.at[pl.ds(NSC * STR, TAIL)],
                        part_hbm.at[c, pl.ds(NSC * STR, TAIL)])
    if with_deg:
        pltpu.sync_copy(dacc.at[pl.ds(s * DSTR, DSTR)], degp_hbm.at[c, pl.ds(s * DSTR, DSTR)])


def _sc_mesh():
    return plsc.VectorSubcoreMesh(core_axis_name="c", subcore_axis_name="s")


def _sc_pass_deg(z, src, dst, znd, zn, ones_k):
    scratch = (
        [pltpu.VMEM((CK,), jnp.int32)] * (2 * NBUF)
        + [pltpu.VMEM((CK, DM), jnp.float32)] * NBUF
        + [pltpu.VMEM((CK,), jnp.float32),
           pltpu.VMEM_SHARED((N, DM), jnp.float32),
           pltpu.VMEM_SHARED((NP,), jnp.float32)]
        + [pltpu.SemaphoreType.DMA] * 4
    )
    f = pl.kernel(
        functools.partial(_sc_body, True),
        out_type=(jax.ShapeDtypeStruct((NC, N, DM), jnp.float32),
                  jax.ShapeDtypeStruct((NC, NP), jnp.float32)),
        mesh=_sc_mesh(),
        scratch_types=scratch,
    )
    return f(z, src, dst, znd, zn, ones_k)


def _sc_pass(z, src, dst, znd):
    scratch = (
        [pltpu.VMEM((CK,), jnp.int32)] * (2 * NBUF)
        + [pltpu.VMEM((CK, DM), jnp.float32)] * NBUF
        + [pltpu.VMEM_SHARED((N, DM), jnp.float32)]
        + [pltpu.SemaphoreType.DMA] * 3
    )
    f = pl.kernel(
        functools.partial(_sc_body, False),
        out_type=jax.ShapeDtypeStruct((NC, N, DM), jnp.float32),
        mesh=_sc_mesh(),
        scratch_types=scratch,
    )
    return f(z, src, dst, znd)


def _pool(h):
    return jnp.max(h.reshape(GB, NB, DM), axis=1)[None]


def _k0_body(geo_ref, sem_ref, a_ref, ts_ref, p_ref, c_ref, wb_ref,
             h_ref, z_ref, g_ref):
    onehot = (sem_ref[...] == lax.broadcasted_iota(
        jnp.int32, (BLK, NS_SEM), 1)).astype(jnp.float32)
    h = (geo_ref[...] @ a_ref[...] + onehot @ ts_ref[...]
         + p_ref[...] + c_ref[...])
    h = jnp.maximum(h, 0.0)
    h_ref[...] = h
    z_ref[...] = h @ wb_ref[...]
    g_ref[...] = _pool(h)


def _gspec():
    return pl.BlockSpec((1, GB, DM), lambda i: (i, 0, 0))


def _gshape():
    return jax.ShapeDtypeStruct((GRID, GB, DM), jnp.float32)


def _tc_input_layer(geo_p, sem2, a_p, ts_p, p_tile, cvec, wb1):
    full = lambda shp: pl.BlockSpec(shp, lambda i: (0, 0))
    return pl.pallas_call(
        _k0_body,
        grid=(GRID,),
        in_specs=[
            pl.BlockSpec((BLK, 8), lambda i: (i, 0)),
            pl.BlockSpec((BLK, 1), lambda i: (i, 0)),
            full((8, DM)),
            full((NS_SEM, DM)),
            full((BLK, DM)),
            full((1, DM)),
            full((DM, DM)),
        ],
        out_specs=[
            pl.BlockSpec((BLK, DM), lambda i: (i, 0)),
            pl.BlockSpec((BLK, DM), lambda i: (i, 0)),
            _gspec(),
        ],
        out_shape=[
            jax.ShapeDtypeStruct((N, DM), jnp.float32),
            jax.ShapeDtypeStruct((N, DM), jnp.float32),
            _gshape(),
        ],
    )(geo_p, sem2, a_p, ts_p, p_tile, cvec, wb1)


def _combine(h_ref, p0_ref, p1_ref, d0_ref, d1_ref, wt_ref, b_ref):
    dsum = d0_ref[...] + d1_ref[...]
    ssum = p0_ref[...] + p1_ref[...]
    inv = 1.0 / jnp.maximum(dsum, 1.0)
    pre = h_ref[...] @ wt_ref[...] + ssum * inv + b_ref[...]
    return jnp.maximum(jnp.where(dsum > 0.0, pre, 0.0), 0.0)


def _klayer_body(h_ref, p0_ref, p1_ref, d0_ref, d1_ref,
                 wt_ref, b_ref, wb_ref, hout_ref, zout_ref, g_ref):
    hnew = _combine(h_ref, p0_ref, p1_ref, d0_ref, d1_ref, wt_ref, b_ref)
    hout_ref[...] = hnew
    zout_ref[...] = hnew @ wb_ref[...]
    g_ref[...] = _pool(hnew)


def _tc_layer(h, p0, p1, d0, d1, wt, b, wb):
    full2 = lambda shp: pl.BlockSpec(shp, lambda i: (0, 0))
    nblk = lambda: pl.BlockSpec((BLK, DM), lambda i: (i, 0))
    return pl.pallas_call(
        _klayer_body,
        grid=(GRID,),
        in_specs=[
            nblk(), nblk(), nblk(),
            pl.BlockSpec((BLK, 1), lambda i: (i, 0)),
            pl.BlockSpec((BLK, 1), lambda i: (i, 0)),
            full2((DM, DM)),
            full2((1, DM)),
            full2((DM, DM)),
        ],
        out_specs=[nblk(), nblk(), _gspec()],
        out_shape=[
            jax.ShapeDtypeStruct((N, DM), jnp.float32),
            jax.ShapeDtypeStruct((N, DM), jnp.float32),
            _gshape(),
        ],
    )(h, p0, p1, d0, d1, wt, b, wb)


def _k3_body(h_ref, p0_ref, p1_ref, d0_ref, d1_ref, wt_ref, b_ref,
             g0_ref, g1_ref, g2_ref, w0_ref, w1_ref, w2_ref, w3_ref,
             bagg_ref, wmu_ref, bmu_ref, wvar_ref, bvar_ref,
             mu_ref, lv_ref):
    h3 = _combine(h_ref, p0_ref, p1_ref, d0_ref, d1_ref, wt_ref, b_ref)
    g3 = _pool(h3)[0]
    latent = (g0_ref[0] @ w0_ref[...] + g1_ref[0] @ w1_ref[...]
              + g2_ref[0] @ w2_ref[...] + g3 @ w3_ref[...] + bagg_ref[...])
    mu_ref[...] = (latent @ wmu_ref[...] + bmu_ref[...])[None]
    lv_ref[...] = (latent @ wvar_ref[...] + bvar_ref[...])[None]


def _tc_final_layer(h, p0, p1, d0, d1, wt, b, g0, g1, g2,
                    w0, w1, w2, w3, bagg2, Wmu, bmu2, Wvar, bvar2):
    full2 = lambda shp: pl.BlockSpec(shp, lambda i: (0, 0))
    nblk = lambda: pl.BlockSpec((BLK, DM), lambda i: (i, 0))
    mu, lv = pl.pallas_call(
        _k3_body,
        grid=(GRID,),
        in_specs=[
            nblk(), nblk(), nblk(),
            pl.BlockSpec((BLK, 1), lambda i: (i, 0)),
            pl.BlockSpec((BLK, 1), lambda i: (i, 0)),
            full2((DM, DM)),
            full2((1, DM)),
            _gspec(), _gspec(), _gspec(),
            full2((DM, DM)), full2((DM, DM)), full2((DM, DM)), full2((DM, DM)),
            full2((1, DM)),
            full2((DM, DM)), full2((1, DM)),
            full2((DM, DM)), full2((1, DM)),
        ],
        out_specs=[_gspec(), _gspec()],
        out_shape=[_gshape(), _gshape()],
    )(h, p0, p1, d0, d1, wt, b, g0, g1, g2,
      w0, w1, w2, w3, bagg2, Wmu, bmu2, Wvar, bvar2)
    return mu.reshape(B, DM), lv.reshape(B, DM)


def kernel(geometry, semantic, edge_index, batch, ptr, Wg, bg, emb, Wlot,
           blot, W1, b1, W2, b2, W3, b3, Wagg, bagg, Wmu, bmu, Wvar, bvar):
    f32 = jnp.float32
    a_p = jnp.pad(Wg @ Wlot[:DM], ((0, 3), (0, 0)))
    ts_p = jnp.pad(emb @ Wlot[DM:2 * DM], ((0, 5), (0, 0)))
    p_tile = jnp.tile(Wlot[2 * DM:], (GB, 1))
    cvec = (bg @ Wlot[:DM] + blot)[None, :]
    geo_p = jnp.pad(geometry, ((0, 0), (0, 3)))
    sem2 = semantic.reshape(N, 1)
    src = edge_index[0]
    dst = edge_index[1]
    znd = jnp.zeros((N, DM), f32)
    zn = jnp.zeros((NP,), f32)
    ones_k = jnp.ones((CK,), f32)

    h0, z1, g0 = _tc_input_layer(geo_p, sem2, a_p, ts_p, p_tile, cvec, W1[DM:])

    part1, degp = _sc_pass_deg(z1, src, dst, znd, zn, ones_k)
    d0 = degp[0, :N].reshape(N, 1)
    d1 = degp[1, :N].reshape(N, 1)

    h1, z2, g1 = _tc_layer(h0, part1[0], part1[1], d0, d1,
                           W1[:DM], b1[None, :], W2[DM:])
    part2 = _sc_pass(z2, src, dst, znd)
    h2, z3, g2 = _tc_layer(h1, part2[0], part2[1], d0, d1,
                           W2[:DM], b2[None, :], W3[DM:])
    part3 = _sc_pass(z3, src, dst, znd)
    mu, lv = _tc_final_layer(h2, part3[0], part3[1], d0, d1,
                             W3[:DM], b3[None, :], g0, g1, g2,
                             Wagg[:DM], Wagg[DM:2 * DM], Wagg[2 * DM:3 * DM],
                             Wagg[3 * DM:], bagg[None, :], Wmu, bmu[None, :],
                             Wvar, bvar[None, :])
    return (mu, lv)
